# baseline (speedup 1.0000x reference)
import functools

import jax
import jax.numpy as jnp
from jax import lax
from jax.experimental import pallas as pl
from jax.experimental.pallas import tpu as pltpu
from jax.experimental.pallas import tpu_sc as plsc

EPS = 1e-8
NUM_CLASSES = 150
CHANNELS = 192
BATCH = 8
HW = 128 * 128
N_PIX = BATCH * HW

NC = 2
NS = 16
NW = NC * NS
LANES = 16

K_PAD = 160
CPT = CHANNELS // NW
ROWS_PER_TILE = (BATCH * 128) // NS
CNT_SLICE = N_PIX // NW
QTR = HW // 4


def _sc_segment_sums(label, feat3):
    mesh = plsc.VectorSubcoreMesh(core_axis_name="c", subcore_axis_name="s")

    @functools.partial(
        pl.kernel,
        out_type=(
            jax.ShapeDtypeStruct((CHANNELS, K_PAD), jnp.float32),
            jax.ShapeDtypeStruct((NW, K_PAD), jnp.float32),
        ),
        mesh=mesh,
        compiler_params=pltpu.CompilerParams(use_tc_tiling_on_sc=False,
                                             needs_layout_passes=False),
        scratch_types=[
            pltpu.VMEM((32 * 512,), jnp.int32),
            pltpu.VMEM((32 * 512,), jnp.int32),
            pltpu.VMEM((ROWS_PER_TILE * 128,), jnp.int32),
            pltpu.VMEM_SHARED((N_PIX,), jnp.int32),
            pltpu.VMEM((QTR,), jnp.int32),
            pltpu.VMEM((QTR,), jnp.int32),
            pltpu.VMEM((QTR,), jnp.int32),
            pltpu.VMEM((CPT, QTR), jnp.float32),
            pltpu.VMEM((CPT, QTR), jnp.float32),
            pltpu.VMEM((CPT, LANES * K_PAD), jnp.float32),
            pltpu.VMEM((CPT, K_PAD), jnp.float32),
            pltpu.VMEM((LANES * K_PAD,), jnp.float32),
            pltpu.VMEM((K_PAD,), jnp.float32),
            pltpu.SemaphoreType.DMA,
            pltpu.SemaphoreType.DMA,
            pltpu.SemaphoreType.DMA,
            pltpu.SemaphoreType.DMA,
            pltpu.SemaphoreType.DMA,
        ],
    )
    def sc_main(label_hbm, feat_hbm, sum_out, cnt_out,
                row_blk0, row_blk1, lab_stage, lab_shared, labq0, labq1,
                cnt_lab, feat_buf0, feat_buf1,
                acc, acc_out, cnt_acc, cnt_vec, sem0, sem1, lsem0, lsem1,
                rsem):
        cid = lax.axis_index("c")
        sid = lax.axis_index("s")
        gwid = cid * NS + sid

        zeros16 = jnp.zeros((LANES,), jnp.float32)
        ones16 = jnp.ones((LANES,), jnp.float32)
        iota16 = lax.iota(jnp.int32, LANES)
        lane_off = iota16 * K_PAD

        b0 = sid // 2
        r0 = (sid % 2) * ROWS_PER_TILE
        row_blks = (row_blk0, row_blk1)

        def _row_src(chunk):
            off = (b0 * 512 + (r0 + chunk * 8) * 4) * 512
            return label_hbm.at[pl.ds(off, 32 * 512)]

        rdescs = [pltpu.async_copy(_row_src(half), row_blks[half], rsem)
                  for half in range(2)]
        for chunk in range(8):
            rdescs[chunk % 2].wait()
            rb = row_blks[chunk % 2]

            @plsc.parallel_loop(0, 8, unroll=2)
            def _down(rr, chunk=chunk, rb=rb):
                for j in range(8):
                    ix = iota16 * 4 + (j * 64) + rr * 2048
                    v = plsc.load_gather(rb, [ix])
                    lab_stage[pl.ds((chunk * 8 + rr) * 128 + j * LANES,
                                    LANES)] = v + lane_off

            if chunk + 2 < 8:
                rdescs[chunk % 2] = pltpu.async_copy(
                    _row_src(chunk + 2), row_blks[chunk % 2], rsem)

        pltpu.sync_copy(
            lab_stage, lab_shared.at[pl.ds(sid * (ROWS_PER_TILE * 128),
                                           ROWS_PER_TILE * 128)])
        plsc.subcore_barrier()

        ch_base = gwid * CPT
        feat_bufs = (feat_buf0, feat_buf1)
        labqs = (labq0, labq1)
        sems = (sem0, sem1)
        n_passes = BATCH * 4

        lsems = (lsem0, lsem1)

        def _start(p):
            b, h = divmod(p, 4)
            fb, lq = feat_bufs[p % 2], labqs[p % 2]
            ds_pix = pl.ds(h * QTR, QTR)
            ds_lab = pl.ds(b * HW + h * QTR, QTR)
            return [
                pltpu.async_copy(
                    feat_hbm.at[b, pl.ds(ch_base, CPT), ds_pix],
                    fb, sems[p % 2]),
                pltpu.async_copy(lab_shared.at[ds_lab], lq, lsems[p % 2]),
            ]

        descs = [_start(0), None]

        @pl.loop(0, (LANES * K_PAD) // LANES, unroll=8)
        def _zc(i):
            cnt_acc[pl.ds(i * LANES, LANES)] = zeros16

        pltpu.sync_copy(lab_shared.at[pl.ds(gwid * CNT_SLICE, CNT_SLICE)],
                        cnt_lab)

        @plsc.parallel_loop(0, CNT_SLICE // LANES, unroll=8)
        def _cnt(i):
            ix = cnt_lab[pl.ds(i * LANES, LANES)]
            plsc.addupdate_scatter(cnt_acc, [ix], ones16)

        @pl.loop(0, K_PAD // LANES)
        def _credu(i):
            tot = zeros16
            for l in range(LANES):
                tot = tot + cnt_acc[pl.ds(l * K_PAD + i * LANES, LANES)]
            cnt_vec[pl.ds(i * LANES, LANES)] = tot

        pltpu.sync_copy(cnt_vec, cnt_out.at[gwid])

        for cl in range(CPT):
            accl = acc.at[cl]

            @pl.loop(0, (LANES * K_PAD) // LANES, unroll=8)
            def _za(i, accl=accl):
                accl[pl.ds(i * LANES, LANES)] = zeros16

        for p in range(n_passes):
            if p + 1 < n_passes:
                descs[(p + 1) % 2] = _start(p + 1)
            for d in descs[p % 2]:
                d.wait()
            fb, lq = feat_bufs[p % 2], labqs[p % 2]
            accs = [acc.at[j] for j in range(CPT)]

            @plsc.parallel_loop(0, QTR // LANES, unroll=4)
            def _seg(i, fb=fb, lq=lq, accs=accs):
                ix = lq[pl.ds(i * LANES, LANES)]
                for j in range(CPT):
                    v = fb[j, pl.ds(i * LANES, LANES)]
                    plsc.addupdate_scatter(accs[j], [ix], v)

        for cl in range(CPT):
            accl = acc.at[cl]
            outl = acc_out.at[cl]

            @pl.loop(0, K_PAD // LANES)
            def _aredu(i, accl=accl, outl=outl):
                tot = zeros16
                for l in range(LANES):
                    tot = tot + accl[pl.ds(l * K_PAD + i * LANES, LANES)]
                outl[pl.ds(i * LANES, LANES)] = tot

        pltpu.sync_copy(acc_out, sum_out.at[pl.ds(ch_base, CPT)])

    return sc_main(label, feat3)


def _epilogue(sum_pad, cnt_pad, cenT_pad):
    def body(sum_ref, cnt_ref, cen_ref, out_ref):
        cnt = jnp.sum(cnt_ref[...], axis=0, keepdims=True)
        present = cnt > 0.0
        cnt_safe = jnp.where(present, cnt, 1.0)
        mean = sum_ref[...] / cnt_safe
        cT = cen_ref[...]
        dot = jnp.sum(mean * cT, axis=0, keepdims=True)
        n1 = jnp.maximum(jnp.sqrt(jnp.sum(mean * mean, axis=0, keepdims=True)), EPS)
        n2 = jnp.maximum(jnp.sqrt(jnp.sum(cT * cT, axis=0, keepdims=True)), EPS)
        cos = jnp.where(present, dot / (n1 * n2), 0.0)
        npres = jnp.sum(present.astype(jnp.float32))
        loss = 1.0 - jnp.sum(cos) / npres
        out_ref[...] = loss[None, None]

    return pl.pallas_call(
        body,
        out_shape=jax.ShapeDtypeStruct((1, 1), jnp.float32),
    )(sum_pad, cnt_pad, cenT_pad)


def kernel(feature, label, centers):
    feat3 = feature.reshape(BATCH, CHANNELS, HW)
    if label.ndim == 4:
        label = label[:, 0]
    sum_pad, cnt_pad = _sc_segment_sums(label.reshape(-1), feat3)
    cenT_pad = jnp.pad(centers.T, ((0, 0), (0, K_PAD - NUM_CLASSES)))
    loss = _epilogue(sum_pad, cnt_pad, cenT_pad)
    return loss[0, 0]


# speedup vs baseline: 1.0006x; 1.0006x over previous
import functools

import jax
import jax.numpy as jnp
from jax import lax
from jax.experimental import pallas as pl
from jax.experimental.pallas import tpu as pltpu
from jax.experimental.pallas import tpu_sc as plsc

EPS = 1e-8
NUM_CLASSES = 150
CHANNELS = 192
BATCH = 8
HW = 128 * 128
N_PIX = BATCH * HW

NC = 2
NS = 16
NW = NC * NS
LANES = 16

K_PAD = 160
CPT = CHANNELS // NW
ROWS_PER_TILE = (BATCH * 128) // NS
CNT_SLICE = N_PIX // NW
QTR = HW // 4


def _sc_segment_sums(label, feat3):
    mesh = plsc.VectorSubcoreMesh(core_axis_name="c", subcore_axis_name="s")

    @functools.partial(
        pl.kernel,
        out_type=(
            jax.ShapeDtypeStruct((CHANNELS * K_PAD,), jnp.float32),
            jax.ShapeDtypeStruct((NW * K_PAD,), jnp.float32),
        ),
        mesh=mesh,
        compiler_params=pltpu.CompilerParams(use_tc_tiling_on_sc=False,
                                             needs_layout_passes=False),
        scratch_types=[
            pltpu.VMEM((32 * 512,), jnp.int32),
            pltpu.VMEM((32 * 512,), jnp.int32),
            pltpu.VMEM((ROWS_PER_TILE * 128,), jnp.int32),
            pltpu.VMEM_SHARED((N_PIX,), jnp.int32),
            pltpu.VMEM((QTR,), jnp.int32),
            pltpu.VMEM((QTR,), jnp.int32),
            pltpu.VMEM((QTR,), jnp.int32),
            pltpu.VMEM((CPT, QTR), jnp.float32),
            pltpu.VMEM((CPT, QTR), jnp.float32),
            pltpu.VMEM((CPT, LANES * K_PAD), jnp.float32),
            pltpu.VMEM((CPT * K_PAD,), jnp.float32),
            pltpu.VMEM((LANES * K_PAD,), jnp.float32),
            pltpu.VMEM((K_PAD,), jnp.float32),
            pltpu.SemaphoreType.DMA,
            pltpu.SemaphoreType.DMA,
            pltpu.SemaphoreType.DMA,
            pltpu.SemaphoreType.DMA,
            pltpu.SemaphoreType.DMA,
        ],
    )
    def sc_main(label_hbm, feat_hbm, sum_out, cnt_out,
                row_blk0, row_blk1, lab_stage, lab_shared, labq0, labq1,
                cnt_lab, feat_buf0, feat_buf1,
                acc, acc_out, cnt_acc, cnt_vec, sem0, sem1, lsem0, lsem1,
                rsem):
        cid = lax.axis_index("c")
        sid = lax.axis_index("s")
        gwid = cid * NS + sid

        zeros16 = jnp.zeros((LANES,), jnp.float32)
        ones16 = jnp.ones((LANES,), jnp.float32)
        iota16 = lax.iota(jnp.int32, LANES)
        lane_off = iota16 * K_PAD

        b0 = sid // 2
        r0 = (sid % 2) * ROWS_PER_TILE
        row_blks = (row_blk0, row_blk1)

        def _row_src(chunk):
            off = (b0 * 512 + (r0 + chunk * 8) * 4) * 512
            return label_hbm.at[pl.ds(off, 32 * 512)]

        rdescs = [pltpu.async_copy(_row_src(half), row_blks[half], rsem)
                  for half in range(2)]
        for chunk in range(8):
            rdescs[chunk % 2].wait()
            rb = row_blks[chunk % 2]

            @plsc.parallel_loop(0, 8, unroll=2)
            def _down(rr, chunk=chunk, rb=rb):
                for j in range(8):
                    ix = iota16 * 4 + (j * 64) + rr * 2048
                    v = plsc.load_gather(rb, [ix])
                    lab_stage[pl.ds((chunk * 8 + rr) * 128 + j * LANES,
                                    LANES)] = v + lane_off

            if chunk + 2 < 8:
                rdescs[chunk % 2] = pltpu.async_copy(
                    _row_src(chunk + 2), row_blks[chunk % 2], rsem)

        pltpu.sync_copy(
            lab_stage, lab_shared.at[pl.ds(sid * (ROWS_PER_TILE * 128),
                                           ROWS_PER_TILE * 128)])
        plsc.subcore_barrier()

        ch_base = gwid * CPT
        feat_bufs = (feat_buf0, feat_buf1)
        labqs = (labq0, labq1)
        sems = (sem0, sem1)
        n_passes = BATCH * 4

        lsems = (lsem0, lsem1)

        def _start(p):
            b, h = divmod(p, 4)
            fb, lq = feat_bufs[p % 2], labqs[p % 2]
            ds_pix = pl.ds(h * QTR, QTR)
            ds_lab = pl.ds(b * HW + h * QTR, QTR)
            return [
                pltpu.async_copy(
                    feat_hbm.at[b, pl.ds(ch_base, CPT), ds_pix],
                    fb, sems[p % 2]),
                pltpu.async_copy(lab_shared.at[ds_lab], lq, lsems[p % 2]),
            ]

        descs = [_start(0), None]

        @pl.loop(0, (LANES * K_PAD) // LANES, unroll=8)
        def _zc(i):
            cnt_acc[pl.ds(i * LANES, LANES)] = zeros16

        pltpu.sync_copy(lab_shared.at[pl.ds(gwid * CNT_SLICE, CNT_SLICE)],
                        cnt_lab)

        @plsc.parallel_loop(0, CNT_SLICE // LANES, unroll=8)
        def _cnt(i):
            ix = cnt_lab[pl.ds(i * LANES, LANES)]
            plsc.addupdate_scatter(cnt_acc, [ix], ones16)

        @pl.loop(0, K_PAD // LANES)
        def _credu(i):
            tot = zeros16
            for l in range(LANES):
                tot = tot + cnt_acc[pl.ds(l * K_PAD + i * LANES, LANES)]
            cnt_vec[pl.ds(i * LANES, LANES)] = tot

        pltpu.sync_copy(cnt_vec, cnt_out.at[pl.ds(gwid * K_PAD, K_PAD)])

        for cl in range(CPT):
            accl = acc.at[cl]

            @pl.loop(0, (LANES * K_PAD) // LANES, unroll=8)
            def _za(i, accl=accl):
                accl[pl.ds(i * LANES, LANES)] = zeros16

        for p in range(n_passes):
            if p + 1 < n_passes:
                descs[(p + 1) % 2] = _start(p + 1)
            for d in descs[p % 2]:
                d.wait()
            fb, lq = feat_bufs[p % 2], labqs[p % 2]
            accs = [acc.at[j] for j in range(CPT)]

            @plsc.parallel_loop(0, QTR // LANES, unroll=4)
            def _seg(i, fb=fb, lq=lq, accs=accs):
                ix = lq[pl.ds(i * LANES, LANES)]
                for j in range(CPT):
                    v = fb[j, pl.ds(i * LANES, LANES)]
                    plsc.addupdate_scatter(accs[j], [ix], v)

        for cl in range(CPT):
            accl = acc.at[cl]

            @pl.loop(0, K_PAD // LANES)
            def _aredu(i, accl=accl, cl=cl):
                tot = zeros16
                for l in range(LANES):
                    tot = tot + accl[pl.ds(l * K_PAD + i * LANES, LANES)]
                acc_out[pl.ds(cl * K_PAD + i * LANES, LANES)] = tot

        pltpu.sync_copy(acc_out,
                        sum_out.at[pl.ds(ch_base * K_PAD, CPT * K_PAD)])

    return sc_main(label, feat3)


def _epilogue(sum_pad, cnt_pad, cenT_pad):
    def body(sum_ref, cnt_ref, cen_ref, out_ref):
        cnt = jnp.sum(cnt_ref[...], axis=0, keepdims=True)
        present = cnt > 0.0
        cnt_safe = jnp.where(present, cnt, 1.0)
        mean = sum_ref[...] / cnt_safe
        cT = cen_ref[...]
        dot = jnp.sum(mean * cT, axis=0, keepdims=True)
        n1 = jnp.maximum(jnp.sqrt(jnp.sum(mean * mean, axis=0, keepdims=True)), EPS)
        n2 = jnp.maximum(jnp.sqrt(jnp.sum(cT * cT, axis=0, keepdims=True)), EPS)
        cos = jnp.where(present, dot / (n1 * n2), 0.0)
        npres = jnp.sum(present.astype(jnp.float32))
        loss = 1.0 - jnp.sum(cos) / npres
        out_ref[...] = loss[None, None]

    return pl.pallas_call(
        body,
        out_shape=jax.ShapeDtypeStruct((1, 1), jnp.float32),
    )(sum_pad, cnt_pad, cenT_pad)


def kernel(feature, label, centers):
    feat3 = feature.reshape(BATCH, CHANNELS, HW)
    if label.ndim == 4:
        label = label[:, 0]
    sum_flat, cnt_flat = _sc_segment_sums(label.reshape(-1), feat3)
    sum_pad = sum_flat.reshape(CHANNELS, K_PAD)
    cnt_pad = cnt_flat.reshape(NW, K_PAD)
    cenT_pad = jnp.pad(centers.T, ((0, 0), (0, K_PAD - NUM_CLASSES)))
    loss = _epilogue(sum_pad, cnt_pad, cenT_pad)
    return loss[0, 0]
